# baseline (speedup 1.0000x reference)
import functools

import jax
import jax.numpy as jnp
from jax import lax
from jax.experimental import pallas as pl
from jax.experimental.pallas import tpu as pltpu
from jax.experimental.pallas import tpu_sc as plsc

_NUM_CORES = 2
_NUM_SUBCORES = 16
_NUM_WORKERS = _NUM_CORES * _NUM_SUBCORES


def _sc_gather(idx_flat, table, chunk):
    (B,) = idx_flat.shape
    V, D = table.shape
    b_per_w = B // _NUM_WORKERS
    n_chunks = b_per_w // chunk
    assert b_per_w % chunk == 0 and B % _NUM_WORKERS == 0

    mesh = plsc.VectorSubcoreMesh(
        core_axis_name="c", subcore_axis_name="s",
        num_cores=_NUM_CORES, num_subcores=_NUM_SUBCORES)

    @functools.partial(
        pl.kernel,
        out_type=jax.ShapeDtypeStruct((B, D), jnp.float32),
        mesh=mesh,
        scratch_types=[
            pltpu.VMEM((chunk,), jnp.int32),
            pltpu.VMEM((chunk, D), jnp.float32),
            pltpu.SemaphoreType.DMA,
        ],
        compiler_params=pltpu.CompilerParams(use_tc_tiling_on_sc=False),
    )
    def emb_kernel(idx_hbm, table_hbm, out_hbm, idx_v, rows_v, sem):
        wid = lax.axis_index("s") * _NUM_CORES + lax.axis_index("c")
        base = wid * b_per_w

        def chunk_body(g, carry):
            off = base + g * chunk
            pltpu.sync_copy(idx_hbm.at[pl.ds(off, chunk)], idx_v)
            pltpu.async_copy(table_hbm.at[idx_v], rows_v, sem).wait()
            pltpu.sync_copy(rows_v, out_hbm.at[pl.ds(off, chunk)])
            return carry

        lax.fori_loop(0, n_chunks, chunk_body, 0)

    return emb_kernel(idx_flat, table)


def _tc_tile_workspace(init_workspace, bs):
    _, W, H = init_workspace.shape
    blk = 256

    def body(ws_ref, out_ref):
        out_ref[...] = jnp.broadcast_to(ws_ref[...], (blk, W, H))

    return pl.pallas_call(
        body,
        grid=(bs // blk,),
        in_specs=[pl.BlockSpec((1, W, H), lambda i: (0, 0, 0))],
        out_specs=pl.BlockSpec((blk, W, H), lambda i: (i, 0, 0)),
        out_shape=jax.ShapeDtypeStruct((bs, W, H), jnp.float32),
    )(init_workspace)


def kernel(input_ids, init_workspace, word_embeddings):
    bs, seq = input_ids.shape
    V, D = word_embeddings.shape
    B = bs * seq
    idx_flat = input_ids.reshape(B).astype(jnp.int32)

    workspace = _tc_tile_workspace(init_workspace, bs)
    rows = _sc_gather(idx_flat, word_embeddings, chunk=1600)
    embeddings = rows.reshape(bs, seq, D)
    return (workspace, embeddings)
